# baseline (speedup 1.0000x reference)
import functools

import jax
import jax.numpy as jnp
from jax import lax
from jax.experimental import pallas as pl
from jax.experimental.pallas import tpu as pltpu
from jax.experimental.pallas import tpu_sc as plsc

B = 4096
FD = 512
GD = 64
CHUNK = 512


def _sc_gather(users, pos, neg, gamma_users, gamma_items, theta_users, beta_items):
    info = plsc.get_sparse_core_info()
    nc, ns = info.num_cores, info.num_subcores
    nw = nc * ns
    bpw = B // nw

    mesh = plsc.VectorSubcoreMesh(core_axis_name="c", subcore_axis_name="s")
    out_type = (
        jax.ShapeDtypeStruct((B, GD), jnp.float32),
        jax.ShapeDtypeStruct((B, GD), jnp.float32),
        jax.ShapeDtypeStruct((B, GD), jnp.float32),
        jax.ShapeDtypeStruct((B, GD), jnp.float32),
        jax.ShapeDtypeStruct((B, 1), jnp.float32),
        jax.ShapeDtypeStruct((B, 1), jnp.float32),
    )

    @functools.partial(
        pl.kernel,
        mesh=mesh,
        out_type=out_type,
        compiler_params=pltpu.CompilerParams(use_tc_tiling_on_sc=False),
        scratch_types=[
            pltpu.VMEM((bpw,), jnp.int32),
            pltpu.VMEM((bpw,), jnp.int32),
            pltpu.VMEM((bpw,), jnp.int32),
            pltpu.VMEM((bpw, GD), jnp.float32),
            pltpu.VMEM((bpw, GD), jnp.float32),
            pltpu.VMEM((bpw, GD), jnp.float32),
            pltpu.VMEM((bpw, GD), jnp.float32),
            pltpu.VMEM((bpw, 1), jnp.float32),
            pltpu.VMEM((bpw, 1), jnp.float32),
            pltpu.SemaphoreType.DMA,
        ],
    )
    def k(users_h, pos_h, neg_h, gu_h, gi_h, tu_h, bi_h,
          ug_h, ut_h, gip_h, gin_h, bp_h, bn_h,
          uidx, pidx, nidx, ug_v, ut_v, gip_v, gin_v, bp_v, bn_v, sem):
        wid = lax.axis_index("s") * nc + lax.axis_index("c")
        base = wid * bpw
        pltpu.sync_copy(users_h.at[pl.ds(base, bpw)], uidx)
        pltpu.sync_copy(pos_h.at[pl.ds(base, bpw)], pidx)
        pltpu.sync_copy(neg_h.at[pl.ds(base, bpw)], nidx)
        c1 = pltpu.async_copy(gu_h.at[uidx], ug_v, sem)
        c2 = pltpu.async_copy(tu_h.at[uidx], ut_v, sem)
        c3 = pltpu.async_copy(gi_h.at[pidx], gip_v, sem)
        c4 = pltpu.async_copy(gi_h.at[nidx], gin_v, sem)
        c5 = pltpu.async_copy(bi_h.at[pidx], bp_v, sem)
        c6 = pltpu.async_copy(bi_h.at[nidx], bn_v, sem)
        c1.wait()
        c2.wait()
        c3.wait()
        c4.wait()
        c5.wait()
        c6.wait()
        pltpu.sync_copy(ug_v, ug_h.at[pl.ds(base, bpw)])
        pltpu.sync_copy(ut_v, ut_h.at[pl.ds(base, bpw)])
        pltpu.sync_copy(gip_v, gip_h.at[pl.ds(base, bpw)])
        pltpu.sync_copy(gin_v, gin_h.at[pl.ds(base, bpw)])
        pltpu.sync_copy(bp_v, bp_h.at[pl.ds(base, bpw)])
        pltpu.sync_copy(bn_v, bn_h.at[pl.ds(base, bpw)])

    return k(users, pos, neg, gamma_users, gamma_items, theta_users, beta_items)


def _tc_body(pos_ref, neg_ref, ug_ref, ut_ref, gip_ref, gin_ref,
             bp_ref, bn_ref, e_ref, bpr_ref, out_ref, b_scr, a_scr):
    i = pl.program_id(0)

    @pl.when(i == 0)
    def _():
        fd = pos_ref[...] - neg_ref[...]
        tid = jnp.dot(fd, e_ref[...], preferred_element_type=jnp.float32)
        s2 = jnp.sum(ut_ref[...] * tid, axis=1)
        s1 = jnp.sum(ug_ref[...] * (gip_ref[...] - gin_ref[...]), axis=1)
        b_scr[...] = s1 + s2
        a_scr[...] = (jnp.dot(fd, bpr_ref[...], preferred_element_type=jnp.float32)
                      + bp_ref[...] - bn_ref[...])

    out_ref[...] = a_scr[pl.ds(i * CHUNK, CHUNK), :] + b_scr[...][None, :]


def _tc_main(pos_f, neg_f, ug, ut, gip, gin, bp, bn, e, beta_prime):
    grid = B // CHUNK
    full2 = lambda i: (0, 0)
    return pl.pallas_call(
        _tc_body,
        grid=(grid,),
        in_specs=[
            pl.BlockSpec((B, FD), full2),
            pl.BlockSpec((B, FD), full2),
            pl.BlockSpec((B, GD), full2),
            pl.BlockSpec((B, GD), full2),
            pl.BlockSpec((B, GD), full2),
            pl.BlockSpec((B, GD), full2),
            pl.BlockSpec((B, 1), full2),
            pl.BlockSpec((B, 1), full2),
            pl.BlockSpec((FD, GD), full2),
            pl.BlockSpec((FD, 1), full2),
        ],
        out_specs=pl.BlockSpec((CHUNK, B), lambda i: (i, 0)),
        out_shape=jax.ShapeDtypeStruct((B, B), jnp.float32),
        scratch_shapes=[
            pltpu.VMEM((B,), jnp.float32),
            pltpu.VMEM((B, 1), jnp.float32),
        ],
    )(pos_f, neg_f, ug, ut, gip, gin, bp, bn, e, beta_prime)


def kernel(users, pos_items, neg_items, pos_items_features, neg_items_features,
           gamma_users, gamma_items, theta_users, E, beta_items, beta_prime):
    users = users.astype(jnp.int32)
    pos_items = pos_items.astype(jnp.int32)
    neg_items = neg_items.astype(jnp.int32)
    ug, ut, gip, gin, bp, bn = _sc_gather(
        users, pos_items, neg_items, gamma_users, gamma_items, theta_users,
        beta_items)
    return _tc_main(pos_items_features, neg_items_features, ug, ut, gip, gin,
                    bp, bn, E, beta_prime)


# speedup vs baseline: 1.5622x; 1.5622x over previous
import functools

import jax
import jax.numpy as jnp
from jax import lax
from jax.experimental import pallas as pl
from jax.experimental.pallas import tpu as pltpu
from jax.experimental.pallas import tpu_sc as plsc

B = 4096
FD = 512
GD = 64
CHUNK = 512


def _sc_gather(users, pos, neg, gamma_users, gamma_items, theta_users, beta_items):
    info = plsc.get_sparse_core_info()
    nc, ns = info.num_cores, info.num_subcores
    nw = nc * ns
    bpw = B // nw

    mesh = plsc.VectorSubcoreMesh(core_axis_name="c", subcore_axis_name="s")
    out_type = (
        jax.ShapeDtypeStruct((B, GD), jnp.float32),
        jax.ShapeDtypeStruct((B, GD), jnp.float32),
        jax.ShapeDtypeStruct((B, GD), jnp.float32),
        jax.ShapeDtypeStruct((B, GD), jnp.float32),
        jax.ShapeDtypeStruct((B, 1), jnp.float32),
        jax.ShapeDtypeStruct((B, 1), jnp.float32),
    )

    @functools.partial(
        pl.kernel,
        mesh=mesh,
        out_type=out_type,
        scratch_types=[
            pltpu.VMEM((bpw,), jnp.int32),
            pltpu.VMEM((bpw,), jnp.int32),
            pltpu.VMEM((bpw,), jnp.int32),
            pltpu.VMEM((bpw, GD), jnp.float32),
            pltpu.VMEM((bpw, GD), jnp.float32),
            pltpu.VMEM((bpw, GD), jnp.float32),
            pltpu.VMEM((bpw, GD), jnp.float32),
            pltpu.VMEM((bpw, 1), jnp.float32),
            pltpu.VMEM((bpw, 1), jnp.float32),
            pltpu.SemaphoreType.DMA,
        ],
    )
    def k(users_h, pos_h, neg_h, gu_h, gi_h, tu_h, bi_h,
          ug_h, ut_h, gip_h, gin_h, bp_h, bn_h,
          uidx, pidx, nidx, ug_v, ut_v, gip_v, gin_v, bp_v, bn_v, sem):
        wid = lax.axis_index("s") * nc + lax.axis_index("c")
        base = wid * bpw
        pltpu.sync_copy(users_h.at[pl.ds(base, bpw)], uidx)
        pltpu.sync_copy(pos_h.at[pl.ds(base, bpw)], pidx)
        pltpu.sync_copy(neg_h.at[pl.ds(base, bpw)], nidx)

        nl = 16

        def fire(c, _):
            b16 = c * nl
            uv = uidx[pl.ds(b16, nl)]
            pv = pidx[pl.ds(b16, nl)]
            nv = nidx[pl.ds(b16, nl)]
            for l in range(nl):
                i = b16 + l
                u = uv[l]
                p = pv[l]
                n = nv[l]
                pltpu.async_copy(gu_h.at[pl.ds(u, 1), :], ug_v.at[pl.ds(i, 1), :], sem)
                pltpu.async_copy(tu_h.at[pl.ds(u, 1), :], ut_v.at[pl.ds(i, 1), :], sem)
                pltpu.async_copy(gi_h.at[pl.ds(p, 1), :], gip_v.at[pl.ds(i, 1), :], sem)
                pltpu.async_copy(gi_h.at[pl.ds(n, 1), :], gin_v.at[pl.ds(i, 1), :], sem)
                pltpu.async_copy(bi_h.at[pl.ds(p, 1), :], bp_v.at[pl.ds(i, 1), :], sem)
                pltpu.async_copy(bi_h.at[pl.ds(n, 1), :], bn_v.at[pl.ds(i, 1), :], sem)
            return _

        lax.fori_loop(0, bpw // nl, fire, None)

        pltpu.make_async_copy(gu_h.at[pl.ds(0, bpw), :], ug_v, sem).wait()
        pltpu.make_async_copy(tu_h.at[pl.ds(0, bpw), :], ut_v, sem).wait()
        pltpu.make_async_copy(gi_h.at[pl.ds(0, bpw), :], gip_v, sem).wait()
        pltpu.make_async_copy(gi_h.at[pl.ds(0, bpw), :], gin_v, sem).wait()
        pltpu.make_async_copy(bi_h.at[pl.ds(0, bpw), :], bp_v, sem).wait()
        pltpu.make_async_copy(bi_h.at[pl.ds(0, bpw), :], bn_v, sem).wait()

        pltpu.sync_copy(ug_v, ug_h.at[pl.ds(base, bpw)])
        pltpu.sync_copy(ut_v, ut_h.at[pl.ds(base, bpw)])
        pltpu.sync_copy(gip_v, gip_h.at[pl.ds(base, bpw)])
        pltpu.sync_copy(gin_v, gin_h.at[pl.ds(base, bpw)])
        pltpu.sync_copy(bp_v, bp_h.at[pl.ds(base, bpw)])
        pltpu.sync_copy(bn_v, bn_h.at[pl.ds(base, bpw)])

    return k(users, pos, neg, gamma_users, gamma_items, theta_users, beta_items)


def _tc_body(pos_ref, neg_ref, ug_ref, ut_ref, gip_ref, gin_ref,
             bp_ref, bn_ref, e_ref, bpr_ref, out_ref, b_scr, a_scr):
    i = pl.program_id(0)

    @pl.when(i == 0)
    def _():
        fd = pos_ref[...] - neg_ref[...]
        tid = jnp.dot(fd, e_ref[...], preferred_element_type=jnp.float32)
        s2 = jnp.sum(ut_ref[...] * tid, axis=1)
        s1 = jnp.sum(ug_ref[...] * (gip_ref[...] - gin_ref[...]), axis=1)
        b_scr[...] = s1 + s2
        a_scr[...] = (jnp.dot(fd, bpr_ref[...], preferred_element_type=jnp.float32)
                      + bp_ref[...] - bn_ref[...])

    out_ref[...] = a_scr[pl.ds(i * CHUNK, CHUNK), :] + b_scr[...][None, :]


def _tc_main(pos_f, neg_f, ug, ut, gip, gin, bp, bn, e, beta_prime):
    grid = B // CHUNK
    full2 = lambda i: (0, 0)
    return pl.pallas_call(
        _tc_body,
        grid=(grid,),
        in_specs=[
            pl.BlockSpec((B, FD), full2),
            pl.BlockSpec((B, FD), full2),
            pl.BlockSpec((B, GD), full2),
            pl.BlockSpec((B, GD), full2),
            pl.BlockSpec((B, GD), full2),
            pl.BlockSpec((B, GD), full2),
            pl.BlockSpec((B, 1), full2),
            pl.BlockSpec((B, 1), full2),
            pl.BlockSpec((FD, GD), full2),
            pl.BlockSpec((FD, 1), full2),
        ],
        out_specs=pl.BlockSpec((CHUNK, B), lambda i: (i, 0)),
        out_shape=jax.ShapeDtypeStruct((B, B), jnp.float32),
        scratch_shapes=[
            pltpu.VMEM((B,), jnp.float32),
            pltpu.VMEM((B, 1), jnp.float32),
        ],
    )(pos_f, neg_f, ug, ut, gip, gin, bp, bn, e, beta_prime)


def kernel(users, pos_items, neg_items, pos_items_features, neg_items_features,
           gamma_users, gamma_items, theta_users, E, beta_items, beta_prime):
    users = users.astype(jnp.int32)
    pos_items = pos_items.astype(jnp.int32)
    neg_items = neg_items.astype(jnp.int32)
    ug, ut, gip, gin, bp, bn = _sc_gather(
        users, pos_items, neg_items, gamma_users, gamma_items, theta_users,
        beta_items)
    return _tc_main(pos_items_features, neg_items_features, ug, ut, gip, gin,
                    bp, bn, E, beta_prime)


# speedup vs baseline: 2.2042x; 1.4110x over previous
import functools

import jax
import jax.numpy as jnp
from jax import lax
from jax.experimental import pallas as pl
from jax.experimental.pallas import tpu as pltpu
from jax.experimental.pallas import tpu_sc as plsc

B = 4096
FD = 512
GD = 64
NU = 1000000
NI = 100000
NS = 16
USPAN = NU // NS
ISPAN = NI // NS
USEG = 62720
USEG15 = 62464
UBASE15 = 937472
UTAIL0 = 999936
UTAILN = NU - UTAIL0
ISEG = 6400
ISEG15 = 6272
IBASE15 = 93696
ITAIL0 = 99968
ITAILN = NI - ITAIL0
CHUNK = 512
NCH = B // 16


def _sc_gather(users, pos, neg, gut, git, tut, bitv, gutail, gitail, btail):
    info = plsc.get_sparse_core_info()
    nc = info.num_cores
    dpc = GD // nc

    mesh = plsc.VectorSubcoreMesh(core_axis_name="c", subcore_axis_name="s")
    out_type = (
        jax.ShapeDtypeStruct((GD, NS, B), jnp.float32),
        jax.ShapeDtypeStruct((GD, NS, B), jnp.float32),
        jax.ShapeDtypeStruct((GD, NS, B), jnp.float32),
        jax.ShapeDtypeStruct((GD, NS, B), jnp.float32),
        jax.ShapeDtypeStruct((NS, B), jnp.float32),
        jax.ShapeDtypeStruct((NS, B), jnp.float32),
    )

    @functools.partial(
        pl.kernel,
        mesh=mesh,
        out_type=out_type,
        compiler_params=pltpu.CompilerParams(needs_layout_passes=False),
        scratch_types=[
            pltpu.VMEM((B,), jnp.int32),
            pltpu.VMEM((B,), jnp.int32),
            pltpu.VMEM((B,), jnp.int32),
            pltpu.VMEM((B,), jnp.int32),
            pltpu.VMEM((B,), jnp.int32),
            pltpu.VMEM((B,), jnp.float32),
            pltpu.VMEM((USEG + 64,), jnp.float32),
            pltpu.VMEM((ISEG + 32,), jnp.float32),
            pltpu.VMEM((GD, UTAILN), jnp.float32),
            pltpu.VMEM((GD, ITAILN), jnp.float32),
            pltpu.VMEM((1, ITAILN), jnp.float32),
            pltpu.SemaphoreType.DMA,
        ],
    )
    def k(users_h, pos_h, neg_h, gut_h, git_h, tut_h, bit_h,
          gutail_h, gitail_h, btail_h,
          xu1_h, xu2_h, xi1_h, xi2_h, xb1_h, xb2_h,
          idx_v, locA, mskA, locB, mskB, vals, useg, iseg,
          utail_v, itail_v, btail_v, sem):
        c = lax.axis_index("c")
        t = lax.axis_index("s")

        ulo = t * USPAN
        ubase = (ulo // 128) * 128
        ubase = pl.multiple_of(ubase, 128)
        ilo = t * ISPAN
        ibase = (ilo // 128) * 128
        ibase = pl.multiple_of(ibase, 128)

        pltpu.sync_copy(gutail_h, utail_v)
        pltpu.sync_copy(gitail_h, itail_v)
        pltpu.sync_copy(btail_h, btail_v)

        def zbody(ch, carry):
            z = idx_v[pl.ds(ch * 16, 16)] * 0
            vals[pl.ds(ch * 16, 16)] = z.astype(jnp.float32)
            return carry
        lax.fori_loop(0, NCH, zbody, None)


        def prep(src_h, loc_ref, msk_ref, span, base, tail0, segoff, lim):
            lo = t * span
            hi = lo + span
            pltpu.sync_copy(src_h, idx_v)

            def pbody(ch, carry):
                v = idx_v[pl.ds(ch * 16, 16)]
                r = v - lo
                oob = lax.shift_right_logical(r | (span - 1 - r), 31)
                msk_ref[pl.ds(ch * 16, 16)] = 1 - oob
                tind = 1 - lax.shift_right_logical(v - tail0, 31)
                lbase = v - base
                lbase = jnp.minimum(jnp.maximum(lbase, 0), lim)
                ltail = jnp.minimum(jnp.maximum(segoff + (v - tail0), 0), lim)
                loc_ref[pl.ds(ch * 16, 16)] = (
                    lbase * (1 - tind) + ltail * tind)
                return carry
            lax.fori_loop(0, NCH, pbody, None)

        def scan_write(seg_ref, loc_ref, msk_ref, out_slice):
            def sbody(ch, carry):
                s = ch * 16
                lv = loc_ref[pl.ds(s, 16)]
                mf = msk_ref[pl.ds(s, 16)].astype(jnp.float32)
                g = plsc.load_gather(seg_ref, [lv])
                old = vals[pl.ds(s, 16)]
                vals[pl.ds(s, 16)] = g * mf + old * (1.0 - mf)
                return carry
            lax.fori_loop(0, NCH, sbody, None)
            pltpu.sync_copy(vals, out_slice)

        def user_phase(tab_h, x_h):
            def dbody(d, carry):
                dg = c * dpc + d

                @pl.when(t < NS - 1)
                def _():
                    pltpu.sync_copy(tab_h.at[dg, pl.ds(ubase, USEG)],
                                    useg.at[pl.ds(0, USEG)])

                @pl.when(t == NS - 1)
                def _():
                    pltpu.sync_copy(tab_h.at[dg, pl.ds(UBASE15, USEG15)],
                                    useg.at[pl.ds(0, USEG15)])

                for kk in range(UTAILN // 16):
                    useg[pl.ds(USEG + kk * 16, 16)] = utail_v[dg, pl.ds(kk * 16, 16)]
                scan_write(useg, locA, mskA, x_h.at[dg, t])
                return carry
            lax.fori_loop(0, dpc, dbody, None)

        def item_stage(tab_row):
            @pl.when(t < NS - 1)
            def _():
                pltpu.sync_copy(tab_row.at[pl.ds(ibase, ISEG)],
                                iseg.at[pl.ds(0, ISEG)])

            @pl.when(t == NS - 1)
            def _():
                pltpu.sync_copy(tab_row.at[pl.ds(IBASE15, ISEG15)],
                                iseg.at[pl.ds(0, ISEG15)])

        prep(users_h, locA, mskA, USPAN, ubase, UTAIL0, USEG, USEG + 63)
        user_phase(gut_h, xu1_h)
        user_phase(tut_h, xu2_h)

        prep(pos_h, locA, mskA, ISPAN, ibase, ITAIL0, ISEG, ISEG + 31)
        prep(neg_h, locB, mskB, ISPAN, ibase, ITAIL0, ISEG, ISEG + 31)

        def ibody(d, carry):
            dg = c * dpc + d
            item_stage(git_h.at[dg])
            for kk in range(ITAILN // 16):
                iseg[pl.ds(ISEG + kk * 16, 16)] = itail_v[dg, pl.ds(kk * 16, 16)]
            scan_write(iseg, locA, mskA, xi1_h.at[dg, t])
            scan_write(iseg, locB, mskB, xi2_h.at[dg, t])
            return carry
        lax.fori_loop(0, dpc, ibody, None)

        @pl.when(c == 0)
        def _():
            item_stage(bit_h.at[0])
            for kk in range(ITAILN // 16):
                iseg[pl.ds(ISEG + kk * 16, 16)] = btail_v[0, pl.ds(kk * 16, 16)]
            scan_write(iseg, locA, mskA, xb1_h.at[t])
            scan_write(iseg, locB, mskB, xb2_h.at[t])

    return k(users, pos, neg, gut, git, tut, bitv, gutail, gitail, btail)


def _tc_merge_body(xu1_ref, xu2_ref, xi1_ref, xi2_ref, xb1_ref, xb2_ref,
                   tu_ref, tp_ref, tn_ref, pos_ref, neg_ref, e_ref, bpr_ref,
                   b_ref, a_ref):
    tu = tu_ref[...]
    tp = tp_ref[...]
    tn = tn_ref[...]
    zc = jnp.zeros((GD, CHUNK), jnp.float32)
    zb = jnp.zeros((1, CHUNK), jnp.float32)
    ug = zc
    ut = zc
    gid = zc
    bpv = zb
    bnv = zb
    for t in range(NS):
        ft = jnp.float32(t)
        ug = ug + jnp.where(tu == ft, xu1_ref[:, t, :], 0.0)
        ut = ut + jnp.where(tu == ft, xu2_ref[:, t, :], 0.0)
        gid = gid + jnp.where(tp == ft, xi1_ref[:, t, :], 0.0)
        gid = gid - jnp.where(tn == ft, xi2_ref[:, t, :], 0.0)
        bpv = bpv + jnp.where(tp == ft, xb1_ref[pl.ds(t, 1), :], 0.0)
        bnv = bnv + jnp.where(tn == ft, xb2_ref[pl.ds(t, 1), :], 0.0)
    fd = pos_ref[...] - neg_ref[...]
    tid = jnp.dot(fd, e_ref[...], preferred_element_type=jnp.float32)
    s2 = jnp.sum(ut * jnp.transpose(tid), axis=0)
    s1 = jnp.sum(ug * gid, axis=0)
    b_ref[...] = s1 + s2
    a_ref[...] = (jnp.dot(fd, bpr_ref[...], preferred_element_type=jnp.float32)
                  + jnp.transpose(bpv) - jnp.transpose(bnv))


def _tc_merge(xu1, xu2, xi1, xi2, xb1, xb2, tu, tp, tn, pos_f, neg_f, e, bpr):
    grid = B // CHUNK
    x3 = pl.BlockSpec((GD, NS, CHUNK), lambda i: (0, 0, i))
    x2 = pl.BlockSpec((NS, CHUNK), lambda i: (0, i))
    tmap = pl.BlockSpec((1, CHUNK), lambda i: (0, i))
    feat = pl.BlockSpec((CHUNK, FD), lambda i: (i, 0))
    return pl.pallas_call(
        _tc_merge_body,
        grid=(grid,),
        in_specs=[x3, x3, x3, x3, x2, x2, tmap, tmap, tmap, feat, feat,
                  pl.BlockSpec((FD, GD), lambda i: (0, 0)),
                  pl.BlockSpec((FD, 1), lambda i: (0, 0))],
        out_specs=(pl.BlockSpec((CHUNK,), lambda i: (i,)),
                   pl.BlockSpec((CHUNK, 1), lambda i: (i, 0))),
        out_shape=(jax.ShapeDtypeStruct((B,), jnp.float32),
                   jax.ShapeDtypeStruct((B, 1), jnp.float32)),
    )(xu1, xu2, xi1, xi2, xb1, xb2, tu, tp, tn, pos_f, neg_f, e, bpr)


def _tc_bcast_body(a_ref, b_ref, out_ref):
    out_ref[...] = a_ref[...] + b_ref[...][None, :]


def _tc_bcast(a, b):
    grid = B // CHUNK
    return pl.pallas_call(
        _tc_bcast_body,
        grid=(grid,),
        in_specs=[pl.BlockSpec((CHUNK, 1), lambda i: (i, 0)),
                  pl.BlockSpec((B,), lambda i: (0,))],
        out_specs=pl.BlockSpec((CHUNK, B), lambda i: (i, 0)),
        out_shape=jax.ShapeDtypeStruct((B, B), jnp.float32),
    )(a, b)


def kernel(users, pos_items, neg_items, pos_items_features, neg_items_features,
           gamma_users, gamma_items, theta_users, E, beta_items, beta_prime):
    users = users.astype(jnp.int32)
    pos_items = pos_items.astype(jnp.int32)
    neg_items = neg_items.astype(jnp.int32)
    gut = gamma_users.T
    tut = theta_users.T
    git = gamma_items.T
    bitv = beta_items.T
    gutail = gut[:, UTAIL0:]
    gitail = git[:, ITAIL0:]
    btail = bitv[:, ITAIL0:]
    xu1, xu2, xi1, xi2, xb1, xb2 = _sc_gather(
        users, pos_items, neg_items, gut, git, tut, bitv,
        gutail, gitail, btail)
    tu = (users // USPAN).astype(jnp.float32).reshape(1, B)
    tp = (pos_items // ISPAN).astype(jnp.float32).reshape(1, B)
    tn = (neg_items // ISPAN).astype(jnp.float32).reshape(1, B)
    b, a = _tc_merge(xu1, xu2, xi1, xi2, xb1, xb2, tu, tp, tn,
                     pos_items_features, neg_items_features, E, beta_prime)
    return _tc_bcast(a, b)


# speedup vs baseline: 3.1456x; 1.4271x over previous
import functools

import jax
import jax.numpy as jnp
from jax import lax
from jax.experimental import pallas as pl
from jax.experimental.pallas import tpu as pltpu
from jax.experimental.pallas import tpu_sc as plsc

B = 4096
FD = 512
GD = 64
NU = 1000000
NI = 100000
NS = 16
USPAN = NU // NS
ISPAN = NI // NS
USEG = 62720
USEG15 = 62464
UBASE15 = 937472
UTAIL0 = 999936
UTAILN = NU - UTAIL0
ISEG = 6400
ISEG15 = 6272
IBASE15 = 93696
ITAIL0 = 99968
ITAILN = NI - ITAIL0
CHUNK = 512
NCH = B // 16
NCC = 24


def _sc_gather(users, pos, neg, gut, git, tut, bitv, gutail, gitail, btail):
    info = plsc.get_sparse_core_info()
    nc = info.num_cores
    dpc = GD // nc

    mesh = plsc.VectorSubcoreMesh(core_axis_name="c", subcore_axis_name="s")
    out_type = (
        jax.ShapeDtypeStruct((GD, NS, B), jnp.float32),
        jax.ShapeDtypeStruct((GD, NS, B), jnp.float32),
        jax.ShapeDtypeStruct((GD, NS, B), jnp.float32),
        jax.ShapeDtypeStruct((GD, NS, B), jnp.float32),
        jax.ShapeDtypeStruct((NS, B), jnp.float32),
        jax.ShapeDtypeStruct((NS, B), jnp.float32),
    )

    @functools.partial(
        pl.kernel,
        mesh=mesh,
        out_type=out_type,
        compiler_params=pltpu.CompilerParams(needs_layout_passes=False),
        scratch_types=[
            pltpu.VMEM((B,), jnp.int32),
            pltpu.VMEM((B,), jnp.int32),
            pltpu.VMEM((B,), jnp.int32),
            pltpu.VMEM((B,), jnp.int32),
            pltpu.VMEM((B,), jnp.int32),
            pltpu.VMEM((NCC * 16,), jnp.int32),
            pltpu.VMEM((NCC * 16,), jnp.int32),
            pltpu.VMEM((NCC * 16,), jnp.int32),
            pltpu.VMEM((NCC * 16,), jnp.int32),
            pltpu.VMEM((B + 16,), jnp.float32),
            pltpu.VMEM((USEG + 64,), jnp.float32),
            pltpu.VMEM((ISEG + 32,), jnp.float32),
            pltpu.VMEM((GD, UTAILN), jnp.float32),
            pltpu.VMEM((GD, ITAILN), jnp.float32),
            pltpu.VMEM((1, ITAILN), jnp.float32),
            pltpu.SemaphoreType.DMA,
        ],
    )
    def k(users_h, pos_h, neg_h, gut_h, git_h, tut_h, bit_h,
          gutail_h, gitail_h, btail_h,
          xu1_h, xu2_h, xi1_h, xi2_h, xb1_h, xb2_h,
          idx_v, locA, mskA, locB, mskB, clocA, cslotA, clocB, cslotB,
          vals, useg, iseg,
          utail_v, itail_v, btail_v, sem):
        c = lax.axis_index("c")
        t = lax.axis_index("s")

        ulo = t * USPAN
        ubase = (ulo // 128) * 128
        ubase = pl.multiple_of(ubase, 128)
        ilo = t * ISPAN
        ibase = (ilo // 128) * 128
        ibase = pl.multiple_of(ibase, 128)

        pltpu.sync_copy(gutail_h, utail_v)
        pltpu.sync_copy(gitail_h, itail_v)
        pltpu.sync_copy(btail_h, btail_v)

        def zbody(ch, carry):
            z = idx_v[pl.ds(ch * 16, 16)] * 0
            vals[pl.ds(ch * 16, 16)] = z.astype(jnp.float32)
            return carry
        lax.fori_loop(0, NCH, zbody, None)


        def prep(src_h, loc_ref, msk_ref, span, base, tail0, segoff, lim):
            lo = t * span
            hi = lo + span
            pltpu.sync_copy(src_h, idx_v)

            def pbody(ch, carry):
                v = idx_v[pl.ds(ch * 16, 16)]
                r = v - lo
                oob = lax.shift_right_logical(r | (span - 1 - r), 31)
                msk_ref[pl.ds(ch * 16, 16)] = 1 - oob
                tind = 1 - lax.shift_right_logical(v - tail0, 31)
                lbase = v - base
                lbase = jnp.minimum(jnp.maximum(lbase, 0), lim)
                ltail = jnp.minimum(jnp.maximum(segoff + (v - tail0), 0), lim)
                loc_ref[pl.ds(ch * 16, 16)] = (
                    lbase * (1 - tind) + ltail * tind)
                return carry
            lax.fori_loop(0, NCH, pbody, None)

        def compact(loc_ref, msk_ref, cloc, cslot):
            def fbody(ch, carry):
                z16 = lax.iota(jnp.int32, 16) * 0
                cloc[pl.ds(ch * 16, 16)] = z16
                cslot[pl.ds(ch * 16, 16)] = z16 + B
                return carry
            lax.fori_loop(0, NCC, fbody, None)

            def cbody(ch, cnt):
                s = ch * 16
                mv = msk_ref[pl.ds(s, 16)]
                incl = plsc.cumsum(mv)
                pos = jnp.minimum(cnt + incl - 1, NCC * 16 - 1)
                mb = mv != 0
                plsc.store_scatter(cloc, [pos], loc_ref[pl.ds(s, 16)], mask=mb)
                slot = lax.iota(jnp.int32, 16) + s
                plsc.store_scatter(cslot, [pos], slot, mask=mb)
                return cnt + incl[15]
            lax.fori_loop(0, NCH, cbody, 0)

        def scan_write(seg_ref, cloc, cslot, out_slice):
            def sbody(ch, carry):
                s = ch * 16
                lv = cloc[pl.ds(s, 16)]
                g = plsc.load_gather(seg_ref, [lv])
                sv = cslot[pl.ds(s, 16)]
                plsc.store_scatter(vals, [sv], g)
                return carry
            lax.fori_loop(0, NCC, sbody, None)
            pltpu.sync_copy(vals.at[pl.ds(0, B)], out_slice)

        def user_phase(tab_h, x_h):
            def dbody(d, carry):
                dg = c * dpc + d

                @pl.when(t < NS - 1)
                def _():
                    pltpu.sync_copy(tab_h.at[dg, pl.ds(ubase, USEG)],
                                    useg.at[pl.ds(0, USEG)])

                @pl.when(t == NS - 1)
                def _():
                    pltpu.sync_copy(tab_h.at[dg, pl.ds(UBASE15, USEG15)],
                                    useg.at[pl.ds(0, USEG15)])

                for kk in range(UTAILN // 16):
                    useg[pl.ds(USEG + kk * 16, 16)] = utail_v[dg, pl.ds(kk * 16, 16)]
                scan_write(useg, clocA, cslotA, x_h.at[dg, t])
                return carry
            lax.fori_loop(0, dpc, dbody, None)

        def item_stage(tab_row):
            @pl.when(t < NS - 1)
            def _():
                pltpu.sync_copy(tab_row.at[pl.ds(ibase, ISEG)],
                                iseg.at[pl.ds(0, ISEG)])

            @pl.when(t == NS - 1)
            def _():
                pltpu.sync_copy(tab_row.at[pl.ds(IBASE15, ISEG15)],
                                iseg.at[pl.ds(0, ISEG15)])

        prep(users_h, locA, mskA, USPAN, ubase, UTAIL0, USEG, USEG + 63)
        compact(locA, mskA, clocA, cslotA)
        user_phase(gut_h, xu1_h)
        user_phase(tut_h, xu2_h)

        prep(pos_h, locA, mskA, ISPAN, ibase, ITAIL0, ISEG, ISEG + 31)
        compact(locA, mskA, clocA, cslotA)
        prep(neg_h, locB, mskB, ISPAN, ibase, ITAIL0, ISEG, ISEG + 31)
        compact(locB, mskB, clocB, cslotB)

        def ibody(d, carry):
            dg = c * dpc + d
            item_stage(git_h.at[dg])
            for kk in range(ITAILN // 16):
                iseg[pl.ds(ISEG + kk * 16, 16)] = itail_v[dg, pl.ds(kk * 16, 16)]
            scan_write(iseg, clocA, cslotA, xi1_h.at[dg, t])
            scan_write(iseg, clocB, cslotB, xi2_h.at[dg, t])
            return carry
        lax.fori_loop(0, dpc, ibody, None)

        @pl.when(c == 0)
        def _():
            item_stage(bit_h.at[0])
            for kk in range(ITAILN // 16):
                iseg[pl.ds(ISEG + kk * 16, 16)] = btail_v[0, pl.ds(kk * 16, 16)]
            scan_write(iseg, clocA, cslotA, xb1_h.at[t])
            scan_write(iseg, clocB, cslotB, xb2_h.at[t])

    return k(users, pos, neg, gut, git, tut, bitv, gutail, gitail, btail)


def _tc_merge_body(xu1_ref, xu2_ref, xi1_ref, xi2_ref, xb1_ref, xb2_ref,
                   tu_ref, tp_ref, tn_ref, pos_ref, neg_ref, e_ref, bpr_ref,
                   b_ref, a_ref):
    tu = tu_ref[...]
    tp = tp_ref[...]
    tn = tn_ref[...]
    zc = jnp.zeros((GD, CHUNK), jnp.float32)
    zb = jnp.zeros((1, CHUNK), jnp.float32)
    ug = zc
    ut = zc
    gid = zc
    bpv = zb
    bnv = zb
    for t in range(NS):
        ft = jnp.float32(t)
        ug = ug + jnp.where(tu == ft, xu1_ref[:, t, :], 0.0)
        ut = ut + jnp.where(tu == ft, xu2_ref[:, t, :], 0.0)
        gid = gid + jnp.where(tp == ft, xi1_ref[:, t, :], 0.0)
        gid = gid - jnp.where(tn == ft, xi2_ref[:, t, :], 0.0)
        bpv = bpv + jnp.where(tp == ft, xb1_ref[pl.ds(t, 1), :], 0.0)
        bnv = bnv + jnp.where(tn == ft, xb2_ref[pl.ds(t, 1), :], 0.0)
    fd = pos_ref[...] - neg_ref[...]
    tid = jnp.dot(fd, e_ref[...], preferred_element_type=jnp.float32)
    s2 = jnp.sum(ut * jnp.transpose(tid), axis=0)
    s1 = jnp.sum(ug * gid, axis=0)
    b_ref[...] = s1 + s2
    a_ref[...] = (jnp.dot(fd, bpr_ref[...], preferred_element_type=jnp.float32)
                  + jnp.transpose(bpv) - jnp.transpose(bnv))


def _tc_merge(xu1, xu2, xi1, xi2, xb1, xb2, tu, tp, tn, pos_f, neg_f, e, bpr):
    grid = B // CHUNK
    x3 = pl.BlockSpec((GD, NS, CHUNK), lambda i: (0, 0, i))
    x2 = pl.BlockSpec((NS, CHUNK), lambda i: (0, i))
    tmap = pl.BlockSpec((1, CHUNK), lambda i: (0, i))
    feat = pl.BlockSpec((CHUNK, FD), lambda i: (i, 0))
    return pl.pallas_call(
        _tc_merge_body,
        grid=(grid,),
        in_specs=[x3, x3, x3, x3, x2, x2, tmap, tmap, tmap, feat, feat,
                  pl.BlockSpec((FD, GD), lambda i: (0, 0)),
                  pl.BlockSpec((FD, 1), lambda i: (0, 0))],
        out_specs=(pl.BlockSpec((CHUNK,), lambda i: (i,)),
                   pl.BlockSpec((CHUNK, 1), lambda i: (i, 0))),
        out_shape=(jax.ShapeDtypeStruct((B,), jnp.float32),
                   jax.ShapeDtypeStruct((B, 1), jnp.float32)),
    )(xu1, xu2, xi1, xi2, xb1, xb2, tu, tp, tn, pos_f, neg_f, e, bpr)


def _tc_bcast_body(a_ref, b_ref, out_ref):
    out_ref[...] = a_ref[...] + b_ref[...][None, :]


def _tc_bcast(a, b):
    grid = B // CHUNK
    return pl.pallas_call(
        _tc_bcast_body,
        grid=(grid,),
        in_specs=[pl.BlockSpec((CHUNK, 1), lambda i: (i, 0)),
                  pl.BlockSpec((B,), lambda i: (0,))],
        out_specs=pl.BlockSpec((CHUNK, B), lambda i: (i, 0)),
        out_shape=jax.ShapeDtypeStruct((B, B), jnp.float32),
    )(a, b)


def kernel(users, pos_items, neg_items, pos_items_features, neg_items_features,
           gamma_users, gamma_items, theta_users, E, beta_items, beta_prime):
    users = users.astype(jnp.int32)
    pos_items = pos_items.astype(jnp.int32)
    neg_items = neg_items.astype(jnp.int32)
    gut = gamma_users.T
    tut = theta_users.T
    git = gamma_items.T
    bitv = beta_items.T
    gutail = gut[:, UTAIL0:]
    gitail = git[:, ITAIL0:]
    btail = bitv[:, ITAIL0:]
    xu1, xu2, xi1, xi2, xb1, xb2 = _sc_gather(
        users, pos_items, neg_items, gut, git, tut, bitv,
        gutail, gitail, btail)
    tu = (users // USPAN).astype(jnp.float32).reshape(1, B)
    tp = (pos_items // ISPAN).astype(jnp.float32).reshape(1, B)
    tn = (neg_items // ISPAN).astype(jnp.float32).reshape(1, B)
    b, a = _tc_merge(xu1, xu2, xi1, xi2, xb1, xb2, tu, tp, tn,
                     pos_items_features, neg_items_features, E, beta_prime)
    return _tc_bcast(a, b)
